# baseline (speedup 1.0000x reference)
import functools

import jax
import jax.numpy as jnp
from jax import lax
from jax.experimental import pallas as pl
from jax.experimental.pallas import tpu as pltpu

VOCAB = 1000000
EMBED = 16
N_IDX = 16384

BV = 262144
GRID = (VOCAB + BV - 1) // BV


def _tc_matvec_body(e_ref, wt_ref, b_ref, o_ref):
    emb_row = e_ref[...]
    y = jax.lax.dot_general(
        emb_row, wt_ref[...], (((1,), (0,)), ((), ())),
        preferred_element_type=jnp.float32,
    )
    o_ref[...] = jnp.reshape(y, (BV,)) + b_ref[...]


def kernel(inputs, emb_table, W, b):
    embeds = jnp.take(emb_table, inputs, axis=0).mean(axis=0)
    wt = W.T
    out = pl.pallas_call(
        _tc_matvec_body,
        grid=(GRID,),
        in_specs=[
            pl.BlockSpec((1, EMBED), lambda i: (0, 0)),
            pl.BlockSpec((EMBED, BV), lambda i: (0, i)),
            pl.BlockSpec((BV,), lambda i: (i,)),
        ],
        out_specs=pl.BlockSpec((BV,), lambda i: (i,)),
        out_shape=jax.ShapeDtypeStruct((VOCAB,), jnp.float32),
        compiler_params=pltpu.CompilerParams(
            dimension_semantics=("arbitrary",),
        ),
    )(embeds.reshape(1, EMBED), wt, b)
    return out


# speedup vs baseline: 1.0120x; 1.0120x over previous
import jax
import jax.numpy as jnp
from jax.experimental import pallas as pl
from jax.experimental.pallas import tpu as pltpu

VOCAB = 1000000
EMBED = 16
N_IDX = 16384

BV = 131072
GRID = (VOCAB + BV - 1) // BV


def _tc_body(g_ref, wt_ref, b_ref, o_ref):
    emb_row = jnp.reshape(
        g_ref[...].sum(axis=1) * (1.0 / N_IDX), (1, EMBED)
    )
    y = jax.lax.dot_general(
        emb_row, wt_ref[...], (((1,), (0,)), ((), ())),
        preferred_element_type=jnp.float32,
    )
    o_ref[...] = jnp.reshape(y, (BV,)) + b_ref[...]


def kernel(inputs, emb_table, W, b):
    gathered_t = jnp.take(emb_table, inputs, axis=0).T
    out = pl.pallas_call(
        _tc_body,
        grid=(GRID,),
        in_specs=[
            pl.BlockSpec((EMBED, N_IDX), lambda i: (0, 0)),
            pl.BlockSpec((EMBED, BV), lambda i: (0, i)),
            pl.BlockSpec((BV,), lambda i: (i,)),
        ],
        out_specs=pl.BlockSpec((BV,), lambda i: (i,)),
        out_shape=jax.ShapeDtypeStruct((VOCAB,), jnp.float32),
        compiler_params=pltpu.CompilerParams(
            dimension_semantics=("arbitrary",),
        ),
    )(gathered_t, W.T, b)
    return out
